# baseline (speedup 1.0000x reference)
import functools

import jax
import jax.numpy as jnp
from jax import lax
from jax.experimental import pallas as pl
from jax.experimental.pallas import tpu as pltpu
from jax.experimental.pallas import tpu_sc as plsc

_STREAMS = 8
_HALVES = 2
_TL = 128


def _gram_body(*refs):
    x_refs, g_ref = refs[:-1], refs[-1]
    for s, x_ref in enumerate(x_refs):
        x3 = x_ref[0, 0]
        x = x3.reshape(x3.shape[0], x3.shape[1] * x3.shape[2])
        g = lax.dot_general(x, x, (((1,), (1,)), ((), ())),
                            preferred_element_type=jnp.float32)
        g_ref[s] = g * (1.0 / x.shape[-1])


@functools.lru_cache(maxsize=None)
def _make_gram(B, N, J, T, half, n_half):
    S = _STREAMS
    BNH = (B * N) // n_half
    TC = T // _TL
    base = half * BNH

    def in_map(s):
        return lambda i: ((base + i * S + s) // N, (base + i * S + s) % N,
                          0, 0, 0)

    return pl.pallas_call(
        _gram_body,
        grid=(BNH // S,),
        in_specs=[pl.BlockSpec((1, 1, J, TC, _TL), in_map(s)) for s in range(S)],
        out_specs=pl.BlockSpec((S, J, J), lambda i: (i, 0, 0)),
        out_shape=pltpu.MemorySpace.HBM((BNH, J, J), jnp.float32),
    )


@functools.lru_cache(maxsize=None)
def _make_pair_gather(BN, J, K):
    info = plsc.get_sparse_core_info()
    NC, NS = info.num_cores, info.num_subcores
    NW = NC * NS
    assert BN % NW == 0 and K % 16 == 0
    bn_per_w = BN // NW
    mesh = plsc.VectorSubcoreMesh(core_axis_name="c", subcore_axis_name="s")

    @functools.partial(
        pl.kernel,
        mesh=mesh,
        compiler_params=pltpu.CompilerParams(needs_layout_passes=False),
        out_type=jax.ShapeDtypeStruct((BN * K,), jnp.float32),
        scratch_types=[
            pltpu.VMEM((J, J), jnp.float32),
            pltpu.VMEM((K,), jnp.int32),
            pltpu.VMEM((K,), jnp.int32),
            pltpu.VMEM((K,), jnp.float32),
        ],
    )
    def pair_gather(g_hbm, il_hbm, ir_hbm, out_hbm, g_v, il_v, ir_v, y_v):
        wid = lax.axis_index("s") * NC + lax.axis_index("c")
        pltpu.sync_copy(il_hbm, il_v)
        pltpu.sync_copy(ir_hbm, ir_v)
        for j in range(bn_per_w):
            bn = wid * bn_per_w + j
            pltpu.sync_copy(g_hbm.at[bn], g_v)
            for c in range(K // 16):
                il = il_v[pl.ds(c * 16, 16)]
                ir = ir_v[pl.ds(c * 16, 16)]
                y_v[pl.ds(c * 16, 16)] = plsc.load_gather(g_v, [il, ir])
            pltpu.sync_copy(y_v, out_hbm.at[pl.ds(bn * K, K)])

    return pair_gather


def kernel(sxl, idx_l, idx_r):
    B, N, J, A, T = sxl.shape
    K = idx_l.shape[0]
    BN = B * N
    x5 = sxl.reshape(B, N, J * A, T // _TL, _TL)
    BNH = BN // _HALVES
    ys = []
    for h in range(_HALVES):
        g = _make_gram(B, N, J * A, T, h, _HALVES)(*([x5] * _STREAMS))
        ys.append(_make_pair_gather(BNH, J * A, K)(g, idx_l, idx_r))
    y = jnp.concatenate(ys)
    return y.reshape(B, N, K, 1)


# speedup vs baseline: 1.1467x; 1.1467x over previous
import functools

import jax
import jax.numpy as jnp
from jax import lax
from jax.experimental import pallas as pl
from jax.experimental.pallas import tpu as pltpu
from jax.experimental.pallas import tpu_sc as plsc

_STREAMS = 16
_TL = 128


def _gram_body(*refs):
    x_refs, g_ref = refs[:-1], refs[-1]
    for s, x_ref in enumerate(x_refs):
        x3 = x_ref[0, 0]
        x = x3.reshape(x3.shape[0], x3.shape[1] * x3.shape[2])
        g = lax.dot_general(x, x, (((1,), (1,)), ((), ())),
                            preferred_element_type=jnp.float32)
        g_ref[s] = g * (1.0 / x.shape[-1])


@functools.lru_cache(maxsize=None)
def _make_gram(B, N, J, T):
    S = _STREAMS
    BN = B * N
    TC = T // _TL

    def in_map(s):
        return lambda i: ((i * S + s) // N, (i * S + s) % N, 0, 0, 0)

    return pl.pallas_call(
        _gram_body,
        grid=(BN // S,),
        in_specs=[pl.BlockSpec((1, 1, J, TC, _TL), in_map(s)) for s in range(S)],
        out_specs=pl.BlockSpec((S, J, J), lambda i: (i, 0, 0)),
        out_shape=pltpu.MemorySpace.HBM((BN, J, J), jnp.float32),
    )


@functools.lru_cache(maxsize=None)
def _make_pair_gather(BN, J, K):
    info = plsc.get_sparse_core_info()
    NC, NS = info.num_cores, info.num_subcores
    NW = NC * NS
    assert BN % NW == 0 and K % 16 == 0
    bn_per_w = BN // NW
    mesh = plsc.VectorSubcoreMesh(core_axis_name="c", subcore_axis_name="s")

    @functools.partial(
        pl.kernel,
        mesh=mesh,
        compiler_params=pltpu.CompilerParams(needs_layout_passes=False),
        out_type=jax.ShapeDtypeStruct((BN * K,), jnp.float32),
        scratch_types=[
            pltpu.VMEM((bn_per_w, J, J), jnp.float32),
            pltpu.VMEM((K,), jnp.int32),
            pltpu.VMEM((K,), jnp.int32),
            pltpu.VMEM((bn_per_w * K,), jnp.float32),
        ],
    )
    def pair_gather(g_hbm, il_hbm, ir_hbm, out_hbm, g_v, il_v, ir_v, y_v):
        wid = lax.axis_index("s") * NC + lax.axis_index("c")
        base = wid * bn_per_w
        pltpu.sync_copy(il_hbm, il_v)
        pltpu.sync_copy(ir_hbm, ir_v)
        pltpu.sync_copy(g_hbm.at[pl.ds(base, bn_per_w)], g_v)
        for j in range(bn_per_w):
            jv = jnp.full((16,), j, jnp.int32)
            for c in range(K // 16):
                il = il_v[pl.ds(c * 16, 16)]
                ir = ir_v[pl.ds(c * 16, 16)]
                y_v[pl.ds(j * K + c * 16, 16)] = plsc.load_gather(
                    g_v, [jv, il, ir])
        pltpu.sync_copy(y_v, out_hbm.at[pl.ds(base * K, bn_per_w * K)])

    return pair_gather


def kernel(sxl, idx_l, idx_r):
    B, N, J, A, T = sxl.shape
    K = idx_l.shape[0]
    BN = B * N
    x5 = sxl.reshape(B, N, J * A, T // _TL, _TL)
    g = _make_gram(B, N, J * A, T)(*([x5] * _STREAMS))
    y = _make_pair_gather(BN, J * A, K)(g, idx_l, idx_r)
    return y.reshape(B, N, K, 1)


# speedup vs baseline: 1.1714x; 1.0215x over previous
import functools

import jax
import jax.numpy as jnp
from jax import lax
from jax.experimental import pallas as pl
from jax.experimental.pallas import tpu as pltpu
from jax.experimental.pallas import tpu_sc as plsc

_STREAMS = 16
_TL = 128


def _gram_body(*refs):
    x_refs, g_ref = refs[:-1], refs[-1]
    gs = []
    for x_ref in x_refs:
        x3 = x_ref[0, 0]
        x = x3.reshape(x3.shape[0], x3.shape[1] * x3.shape[2])
        g = lax.dot_general(x, x, (((1,), (1,)), ((), ())),
                            preferred_element_type=jnp.float32)
        gs.append(g * (1.0 / x.shape[-1]))
    for p in range(len(gs) // 2):
        g_ref[p] = lax.concatenate([gs[2 * p], gs[2 * p + 1]], 1)


@functools.lru_cache(maxsize=None)
def _make_gram(B, N, J, T):
    S = _STREAMS
    BN = B * N
    TC = T // _TL

    def in_map(s):
        return lambda i: ((i * S + s) // N, (i * S + s) % N, 0, 0, 0)

    return pl.pallas_call(
        _gram_body,
        grid=(BN // S,),
        in_specs=[pl.BlockSpec((1, 1, J, TC, _TL), in_map(s)) for s in range(S)],
        out_specs=pl.BlockSpec((S // 2, J, 2 * J), lambda i: (i, 0, 0)),
        out_shape=pltpu.MemorySpace.HBM((BN // 2, J, 2 * J), jnp.float32),
    )


@functools.lru_cache(maxsize=None)
def _make_pair_gather(BN, J, K):
    info = plsc.get_sparse_core_info()
    NC, NS = info.num_cores, info.num_subcores
    NW = NC * NS
    assert BN % NW == 0 and K % 16 == 0
    bn_per_w = BN // NW
    mesh = plsc.VectorSubcoreMesh(core_axis_name="c", subcore_axis_name="s")

    @functools.partial(
        pl.kernel,
        mesh=mesh,
        compiler_params=pltpu.CompilerParams(needs_layout_passes=False),
        out_type=jax.ShapeDtypeStruct((BN * K,), jnp.float32),
        scratch_types=[
            pltpu.VMEM((bn_per_w // 2, J, 2 * J), jnp.float32),
            pltpu.VMEM((K,), jnp.int32),
            pltpu.VMEM((K,), jnp.int32),
            pltpu.VMEM((bn_per_w * K,), jnp.float32),
        ],
    )
    def pair_gather(g_hbm, il_hbm, ir_hbm, out_hbm, g_v, il_v, ir_v, y_v):
        wid = lax.axis_index("s") * NC + lax.axis_index("c")
        base = wid * bn_per_w
        pltpu.sync_copy(il_hbm, il_v)
        pltpu.sync_copy(ir_hbm, ir_v)
        pltpu.sync_copy(g_hbm.at[pl.ds(base // 2, bn_per_w // 2)], g_v)
        for j in range(bn_per_w):
            jv = jnp.full((16,), j // 2, jnp.int32)
            off = (j % 2) * J
            for c in range(K // 16):
                il = il_v[pl.ds(c * 16, 16)]
                ir = ir_v[pl.ds(c * 16, 16)]
                y_v[pl.ds(j * K + c * 16, 16)] = plsc.load_gather(
                    g_v, [jv, il, ir + off])
        pltpu.sync_copy(y_v, out_hbm.at[pl.ds(base * K, bn_per_w * K)])

    return pair_gather


def kernel(sxl, idx_l, idx_r):
    B, N, J, A, T = sxl.shape
    K = idx_l.shape[0]
    BN = B * N
    x5 = sxl.reshape(B, N, J * A, T // _TL, _TL)
    g = _make_gram(B, N, J * A, T)(*([x5] * _STREAMS))
    y = _make_pair_gather(BN, J * A, K)(g, idx_l, idx_r)
    return y.reshape(B, N, K, 1)


# speedup vs baseline: 1.1773x; 1.0050x over previous
import functools

import jax
import jax.numpy as jnp
from jax import lax
from jax.experimental import pallas as pl
from jax.experimental.pallas import tpu as pltpu
from jax.experimental.pallas import tpu_sc as plsc

_STREAMS = 32
_TL = 128


def _gram_body(*refs):
    x_refs, g_ref = refs[:-1], refs[-1]
    gs = []
    for x_ref in x_refs:
        x3 = x_ref[0, 0]
        x = x3.reshape(x3.shape[0], x3.shape[1] * x3.shape[2])
        g = lax.dot_general(x, x, (((1,), (1,)), ((), ())),
                            preferred_element_type=jnp.float32)
        gs.append(g * (1.0 / x.shape[-1]))
    for p in range(len(gs) // 2):
        g_ref[p] = lax.concatenate([gs[2 * p], gs[2 * p + 1]], 1)


@functools.lru_cache(maxsize=None)
def _make_gram(B, N, J, T):
    S = _STREAMS
    BN = B * N
    TC = T // _TL

    def in_map(s):
        return lambda i: ((i * S + s) // N, (i * S + s) % N, 0, 0, 0)

    return pl.pallas_call(
        _gram_body,
        grid=(BN // S,),
        in_specs=[pl.BlockSpec((1, 1, J, TC, _TL), in_map(s)) for s in range(S)],
        out_specs=pl.BlockSpec((S // 2, J, 2 * J), lambda i: (i, 0, 0)),
        out_shape=pltpu.MemorySpace.HBM((BN // 2, J, 2 * J), jnp.float32),
    )


@functools.lru_cache(maxsize=None)
def _make_pair_gather(BN, J, K):
    info = plsc.get_sparse_core_info()
    NC, NS = info.num_cores, info.num_subcores
    NW = NC * NS
    assert BN % NW == 0 and K % 16 == 0
    bn_per_w = BN // NW
    mesh = plsc.VectorSubcoreMesh(core_axis_name="c", subcore_axis_name="s")

    @functools.partial(
        pl.kernel,
        mesh=mesh,
        compiler_params=pltpu.CompilerParams(needs_layout_passes=False),
        out_type=jax.ShapeDtypeStruct((BN * K,), jnp.float32),
        scratch_types=[
            pltpu.VMEM((bn_per_w // 2, J, 2 * J), jnp.float32),
            pltpu.VMEM((K,), jnp.int32),
            pltpu.VMEM((K,), jnp.int32),
            pltpu.VMEM((bn_per_w * K,), jnp.float32),
        ],
    )
    def pair_gather(g_hbm, il_hbm, ir_hbm, out_hbm, g_v, il_v, ir_v, y_v):
        wid = lax.axis_index("s") * NC + lax.axis_index("c")
        base = wid * bn_per_w
        pltpu.sync_copy(il_hbm, il_v)
        pltpu.sync_copy(ir_hbm, ir_v)
        pltpu.sync_copy(g_hbm.at[pl.ds(base // 2, bn_per_w // 2)], g_v)
        for j in range(bn_per_w):
            jv = jnp.full((16,), j // 2, jnp.int32)
            off = (j % 2) * J
            for c in range(K // 16):
                il = il_v[pl.ds(c * 16, 16)]
                ir = ir_v[pl.ds(c * 16, 16)]
                y_v[pl.ds(j * K + c * 16, 16)] = plsc.load_gather(
                    g_v, [jv, il, ir + off])
        pltpu.sync_copy(y_v, out_hbm.at[pl.ds(base * K, bn_per_w * K)])

    return pair_gather


def kernel(sxl, idx_l, idx_r):
    B, N, J, A, T = sxl.shape
    K = idx_l.shape[0]
    BN = B * N
    x5 = sxl.reshape(B, N, J * A, T // _TL, _TL)
    g = _make_gram(B, N, J * A, T)(*([x5] * _STREAMS))
    y = _make_pair_gather(BN, J * A, K)(g, idx_l, idx_r)
    return y.reshape(B, N, K, 1)


# speedup vs baseline: 1.1776x; 1.0003x over previous
import functools

import jax
import jax.numpy as jnp
from jax import lax
from jax.experimental import pallas as pl
from jax.experimental.pallas import tpu as pltpu
from jax.experimental.pallas import tpu_sc as plsc

_STREAMS = 32
_TL = 128


def _gram_body(*refs):
    x_refs, g_ref = refs[:-1], refs[-1]
    gs = []
    for x_ref in x_refs:
        x3 = x_ref[0, 0]
        x = x3.reshape(x3.shape[0], x3.shape[1] * x3.shape[2])
        g = lax.dot_general(x, x, (((1,), (1,)), ((), ())),
                            preferred_element_type=jnp.float32)
        gs.append(g * (1.0 / x.shape[-1]))
    for p in range(len(gs) // 2):
        g_ref[p] = lax.concatenate([gs[2 * p], gs[2 * p + 1]], 1)


@functools.lru_cache(maxsize=None)
def _make_gram(B, N, J, T):
    S = _STREAMS
    BN = B * N
    TC = T // _TL

    def in_map(s):
        return lambda i: ((i * S + s) // N, (i * S + s) % N, 0, 0, 0)

    return pl.pallas_call(
        _gram_body,
        grid=(BN // S,),
        in_specs=[pl.BlockSpec((1, 1, J, TC, _TL), in_map(s)) for s in range(S)],
        out_specs=pl.BlockSpec((S // 2, J, 2 * J), lambda i: (i, 0, 0)),
        out_shape=pltpu.MemorySpace.HBM((BN // 2, J, 2 * J), jnp.float32),
    )


@functools.lru_cache(maxsize=None)
def _make_pair_gather(BN, J, K):
    info = plsc.get_sparse_core_info()
    NC, NS = info.num_cores, info.num_subcores
    NW = NC * NS
    assert BN % NW == 0 and K % 16 == 0
    bn_per_w = BN // NW
    mesh = plsc.VectorSubcoreMesh(core_axis_name="c", subcore_axis_name="s")

    @functools.partial(
        pl.kernel,
        mesh=mesh,
        compiler_params=pltpu.CompilerParams(
            needs_layout_passes=False,
            disable_bounds_checks=True,
            disable_semaphore_checks=True,
        ),
        out_type=jax.ShapeDtypeStruct((BN * K,), jnp.float32),
        scratch_types=[
            pltpu.VMEM((bn_per_w // 2, J, 2 * J), jnp.float32),
            pltpu.VMEM((K,), jnp.int32),
            pltpu.VMEM((K,), jnp.int32),
            pltpu.VMEM((bn_per_w * K,), jnp.float32),
        ],
    )
    def pair_gather(g_hbm, il_hbm, ir_hbm, out_hbm, g_v, il_v, ir_v, y_v):
        wid = lax.axis_index("s") * NC + lax.axis_index("c")
        base = wid * bn_per_w
        pltpu.sync_copy(il_hbm, il_v)
        pltpu.sync_copy(ir_hbm, ir_v)
        pltpu.sync_copy(g_hbm.at[pl.ds(base // 2, bn_per_w // 2)], g_v)
        for j in range(bn_per_w):
            jv = jnp.full((16,), j // 2, jnp.int32)
            off = (j % 2) * J
            for c in range(K // 16):
                il = il_v[pl.ds(c * 16, 16)]
                ir = ir_v[pl.ds(c * 16, 16)]
                y_v[pl.ds(j * K + c * 16, 16)] = plsc.load_gather(
                    g_v, [jv, il, ir + off])
        pltpu.sync_copy(y_v, out_hbm.at[pl.ds(base * K, bn_per_w * K)])

    return pair_gather


def kernel(sxl, idx_l, idx_r):
    B, N, J, A, T = sxl.shape
    K = idx_l.shape[0]
    BN = B * N
    x5 = sxl.reshape(B, N, J * A, T // _TL, _TL)
    g = _make_gram(B, N, J * A, T)(*([x5] * _STREAMS))
    y = _make_pair_gather(BN, J * A, K)(g, idx_l, idx_r)
    return y.reshape(B, N, K, 1)


# speedup vs baseline: 1.1787x; 1.0009x over previous
import functools

import jax
import jax.numpy as jnp
from jax import lax
from jax.experimental import pallas as pl
from jax.experimental.pallas import tpu as pltpu
from jax.experimental.pallas import tpu_sc as plsc

_STREAMS = 32
_TL = 128


def _gram_body(*refs):
    x_refs, g_ref = refs[:-1], refs[-1]
    gs = []
    for x_ref in x_refs:
        x3 = x_ref[0, 0]
        x = x3.reshape(x3.shape[0], x3.shape[1] * x3.shape[2])
        g = lax.dot_general(x, x, (((1,), (1,)), ((), ())),
                            preferred_element_type=jnp.float32)
        gs.append(g * (1.0 / x.shape[-1]))
    for p in range(len(gs) // 2):
        g_ref[p] = lax.concatenate([gs[2 * p], gs[2 * p + 1]], 1)


@functools.lru_cache(maxsize=None)
def _make_gram(B, N, J, T):
    S = _STREAMS
    BN = B * N
    TC = T // _TL

    def in_map(s):
        return lambda i: ((i * S + s) // N, (i * S + s) % N, 0, 0, 0)

    return pl.pallas_call(
        _gram_body,
        grid=(BN // S,),
        in_specs=[pl.BlockSpec((1, 1, J, TC, _TL), in_map(s)) for s in range(S)],
        out_specs=pl.BlockSpec((S // 2, J, 2 * J), lambda i: (i, 0, 0)),
        out_shape=pltpu.MemorySpace.HBM((BN // 2, J, 2 * J), jnp.float32),
    )


@functools.lru_cache(maxsize=None)
def _make_pair_gather(BN, J, K):
    info = plsc.get_sparse_core_info()
    NC, NS = info.num_cores, info.num_subcores
    NW = NC * NS
    assert BN % NW == 0 and K % 16 == 0
    bn_per_w = BN // NW
    mesh = plsc.VectorSubcoreMesh(core_axis_name="c", subcore_axis_name="s")

    @functools.partial(
        pl.kernel,
        mesh=mesh,
        compiler_params=pltpu.CompilerParams(needs_layout_passes=False),
        out_type=jax.ShapeDtypeStruct((BN * K,), jnp.float32),
        scratch_types=[
            pltpu.VMEM((bn_per_w // 2, J, 2 * J), jnp.float32),
            pltpu.VMEM((K,), jnp.int32),
            pltpu.VMEM((K,), jnp.int32),
            pltpu.VMEM((bn_per_w * K,), jnp.float32),
        ],
    )
    def pair_gather(g_hbm, il_hbm, ir_hbm, out_hbm, g_v, il_v, ir_v, y_v):
        wid = lax.axis_index("s") * NC + lax.axis_index("c")
        base = wid * bn_per_w
        pltpu.sync_copy(il_hbm, il_v)
        pltpu.sync_copy(ir_hbm, ir_v)
        pltpu.sync_copy(g_hbm.at[pl.ds(base // 2, bn_per_w // 2)], g_v)
        for j in range(bn_per_w):
            jv = jnp.full((16,), j // 2, jnp.int32)
            off = (j % 2) * J
            for c in range(K // 16):
                il = il_v[pl.ds(c * 16, 16)]
                ir = ir_v[pl.ds(c * 16, 16)]
                y_v[pl.ds(j * K + c * 16, 16)] = plsc.load_gather(
                    g_v, [jv, il, ir + off])
        pltpu.sync_copy(y_v, out_hbm.at[pl.ds(base * K, bn_per_w * K)])

    return pair_gather


def kernel(sxl, idx_l, idx_r):
    B, N, J, A, T = sxl.shape
    K = idx_l.shape[0]
    BN = B * N
    x5 = sxl.reshape(B, N, J * A, T // _TL, _TL)
    g = _make_gram(B, N, J * A, T)(*([x5] * _STREAMS))
    y = _make_pair_gather(BN, J * A, K)(g, idx_l, idx_r)
    return y.reshape(B, N, K, 1)
